# baseline (speedup 1.0000x reference)
import functools

import numpy as np
import jax
import jax.numpy as jnp
from jax.experimental import pallas as pl
from jax.experimental.pallas import tpu as pltpu

(_PI, _OI, _OP, _WI, _CC, _RDY, _CB, _SF, _SB, _SI, _SC) = range(11)


def _ft_kernel(x_ref, w_ref, o_ref):
    o_ref[...] = jnp.dot(x_ref[...], w_ref[...],
                         preferred_element_type=jnp.float32)


def _fused_kernel(sched_ref, panel_ref, adj_ref, s1_ref, b1_ref, w2_ref,
                  b2_ref, wl1_ref, bl1_ref, wl2_ref, bl2_ref, wl3_ref,
                  bl3_ref, out_ref, buf_sc, acc_sc, t_sc, stash_sc,
                  sem0, sem1, sem2, *, n, hid, bm1, bm2, cw, nc, region, stash_w):
    t = pl.program_id(0)
    op = sched_ref[_OP, t]
    wi = sched_ref[_WI, t]
    cc = sched_ref[_CC, t]
    cb = sched_ref[_CB, t]

    @pl.when(sched_ref[_SF, t] == 1)
    def _start_next():
        src = adj_ref.at[
            pl.ds(pl.multiple_of(sched_ref[_SI, t] * bm2, bm2), bm2),
            pl.ds(pl.multiple_of(sched_ref[_SC, t] * cw, cw), cw)]

        @pl.when(sched_ref[_SB, t] == 0)
        def _():
            pltpu.make_async_copy(src, buf_sc.at[0], sem0).start()

        @pl.when(sched_ref[_SB, t] == 1)
        def _():
            pltpu.make_async_copy(src, buf_sc.at[1], sem1).start()

        @pl.when(sched_ref[_SB, t] == 2)
        def _():
            pltpu.make_async_copy(src, buf_sc.at[2], sem2).start()

    @pl.when(op == 0)
    def _phase1():
        rdy = sched_ref[_RDY, t]
        row = pl.ds(wi * bm1, bm1)
        acc_sc[row, :] = jnp.zeros((bm1, hid), jnp.float32)
        u = jnp.zeros((bm1, hid), jnp.float32)
        for c in range(nc):
            cs = c * cw
            tbc = panel_ref[:, cs:cs + cw]
            u = u + jnp.dot(tbc, s1_ref[cs:cs + cw, :],
                            precision=jax.lax.Precision.DEFAULT,
                            preferred_element_type=jnp.float32)

            @pl.when((rdy & (1 << c)) != 0)
            def _(tbc=tbc, cs=cs):
                acc_sc[row, :] += jnp.dot(tbc, t_sc[cs:cs + cw, :],
                                          precision=jax.lax.Precision.DEFAULT,
                                          preferred_element_type=jnp.float32)
        if stash_w:
            tbt = panel_ref[:, region:n]
            u = u + jnp.dot(tbt, s1_ref[region:n, :],
                            preferred_element_type=jnp.float32)
            stash_sc[row, :] = tbt
        h1 = jnp.maximum(u + b1_ref[...], 0.0)
        t_sc[row, :] = jnp.dot(h1, w2_ref[...],
                               preferred_element_type=jnp.float32)

    @pl.when(op == 1)
    def _phase2():
        src = adj_ref.at[pl.ds(pl.multiple_of(wi * bm2, bm2), bm2),
                         pl.ds(pl.multiple_of(cc * cw, cw), cw)]

        @pl.when(cb == 0)
        def _():
            pltpu.make_async_copy(src, buf_sc.at[0], sem0).wait()

        @pl.when(cb == 1)
        def _():
            pltpu.make_async_copy(src, buf_sc.at[1], sem1).wait()

        @pl.when(cb == 2)
        def _():
            pltpu.make_async_copy(src, buf_sc.at[2], sem2).wait()

        tb = buf_sc[cb]
        acc_sc[pl.ds(wi * bm2, bm2), :] += jnp.dot(
            tb, t_sc[pl.ds(pl.multiple_of(cc * cw, cw), cw), :],
            precision=jax.lax.Precision.DEFAULT,
            preferred_element_type=jnp.float32)

    @pl.when(op == 2)
    def _epilogue():
        row = pl.ds(wi * bm2, bm2)
        base = acc_sc[row, :]
        if stash_w:
            base = base + jnp.dot(stash_sc[row, :], t_sc[region:n, :],
                                  preferred_element_type=jnp.float32)
        h = jnp.maximum(base + b2_ref[...], 0.0)
        h = jnp.maximum(jnp.dot(h, wl1_ref[...],
                                preferred_element_type=jnp.float32)
                        + bl1_ref[...], 0.0)
        h = jnp.maximum(jnp.dot(h, wl2_ref[...],
                                preferred_element_type=jnp.float32)
                        + bl2_ref[...], 0.0)
        logits = jnp.dot(h, wl3_ref[...],
                         preferred_element_type=jnp.float32) + bl3_ref[...]
        m = jnp.max(logits, axis=-1, keepdims=True)
        lse = m + jnp.log(jnp.sum(jnp.exp(logits - m), axis=-1, keepdims=True))
        out_ref[...] = logits - lse


def _build_schedule(r1, r2, bm1, bm2, cw, nc):
    q = bm2 // bm1
    steps = []
    for i1 in range(r1):
        parent_start = bm2 * (i1 // q)
        mask = 0
        for c in range(nc):
            if (c + 1) * cw <= parent_start:
                mask |= 1 << c
        steps.append([i1, 0, 0, i1, 0, mask])
    for i2 in range(r2):
        done = bm2 * i2
        for c in range(nc):
            if (c + 1) * cw > done:
                steps.append([r1 - 1, i2, 1, i2, c, 0])
        steps.append([r1 - 1, i2, 2, i2, 0, 0])
    s = len(steps)
    sched = np.zeros((11, s), dtype=np.int32)
    for t, (pi, oi, op, wi, cc, rdy) in enumerate(steps):
        sched[_PI, t] = pi
        sched[_OI, t] = oi
        sched[_OP, t] = op
        sched[_WI, t] = wi
        sched[_CC, t] = cc
        sched[_RDY, t] = rdy
    chunk_steps = [t for t in range(s) if sched[_OP, t] == 1]
    for k, tt in enumerate(chunk_steps):
        sched[_CB, tt] = k % 3
        if k == 0:
            site = r1 - 2
        elif k == 1:
            site = r1 - 1
        else:
            site = chunk_steps[k - 2]
        sched[_SF, site] = 1
        sched[_SB, site] = k % 3
        sched[_SI, site] = sched[_WI, tt]
        sched[_SC, site] = sched[_CC, tt]
    return jnp.asarray(sched), s


def _pick_cfg(n):
    if n % 1000 == 0 and n >= 4000:
        bm1, bm2 = 200, 1000
    else:
        bm1 = bm2 = None
        for b in (80, 40, 8):
            if n % b == 0:
                bm1 = b
                break
        bm2 = bm1
        for b in (2 * bm1, 4 * bm1):
            if n % b == 0:
                bm2 = b
    region = (n // 128) * 128
    cw = None
    for c in (768, 512, 256, 128):
        if region % c == 0 and (n < 4000) == (c <= 256):
            cw = c
            break
    if cw is None:
        cw = 128
    return bm1, bm2, cw, region


def kernel(x, adj, W_gc1, b_gc1, W_gc2, b_gc2, W_l1, b_l1, W_l2, b_l2,
           W_l3, b_l3):
    _, n, in_f = x.shape
    hid = W_gc1.shape[1]
    hid2 = W_l2.shape[1]
    classes = W_l3.shape[1]
    x2 = x.reshape(n, in_f)
    adj2 = adj.reshape(n, n)
    bm1, bm2, cw, region = _pick_cfg(n)
    r1, r2 = n // bm1, n // bm2
    nc = region // cw
    stash_w = n - region

    s1 = pl.pallas_call(
        _ft_kernel,
        grid=(r1,),
        in_specs=[
            pl.BlockSpec((bm1, in_f), lambda i: (i, 0)),
            pl.BlockSpec((in_f, hid), lambda i: (0, 0)),
        ],
        out_specs=pl.BlockSpec((bm1, hid), lambda i: (i, 0)),
        out_shape=jax.ShapeDtypeStruct((n, hid), jnp.float32),
    )(x2, W_gc1)

    sched, steps = _build_schedule(r1, r2, bm1, bm2, cw, nc)

    const = lambda t, s: (0, 0)
    grid_spec = pltpu.PrefetchScalarGridSpec(
        num_scalar_prefetch=1,
        grid=(steps,),
        in_specs=[
            pl.BlockSpec((bm1, n), lambda t, s: (s[_PI, t], 0)),
            pl.BlockSpec(memory_space=pl.ANY),
            pl.BlockSpec((n, hid), const),
            pl.BlockSpec((1, hid), const),
            pl.BlockSpec((hid, hid), const),
            pl.BlockSpec((1, hid), const),
            pl.BlockSpec((hid, hid), const),
            pl.BlockSpec((1, hid), const),
            pl.BlockSpec((hid, hid2), const),
            pl.BlockSpec((1, hid2), const),
            pl.BlockSpec((hid2, classes), const),
            pl.BlockSpec((1, classes), const),
        ],
        out_specs=pl.BlockSpec((bm2, classes), lambda t, s: (s[_OI, t], 0)),
        scratch_shapes=[
            pltpu.VMEM((3, bm2, cw), jnp.float32),
            pltpu.VMEM((n, hid), jnp.float32),
            pltpu.VMEM((n, hid), jnp.float32),
            pltpu.VMEM((n, max(stash_w, 1)), jnp.float32),
            pltpu.SemaphoreType.DMA,
            pltpu.SemaphoreType.DMA,
            pltpu.SemaphoreType.DMA,
        ],
    )

    out_nc = pl.pallas_call(
        functools.partial(_fused_kernel, n=n, hid=hid, bm1=bm1, bm2=bm2,
                          cw=cw, nc=nc, region=region, stash_w=stash_w),
        grid_spec=grid_spec,
        out_shape=jax.ShapeDtypeStruct((n, classes), jnp.float32),
        compiler_params=pltpu.CompilerParams(
            dimension_semantics=("arbitrary",)),
    )(sched, adj2, adj2, s1, b_gc1.reshape(1, hid), W_gc2,
      b_gc2.reshape(1, hid), W_l1, b_l1.reshape(1, hid),
      W_l2, b_l2.reshape(1, hid2), W_l3, b_l3.reshape(1, classes))

    return jnp.transpose(out_nc)[None]


# speedup vs baseline: 1.0219x; 1.0219x over previous
import functools

import numpy as np
import jax
import jax.numpy as jnp
from jax.experimental import pallas as pl
from jax.experimental.pallas import tpu as pltpu

(_PI, _OI, _OP, _WI, _CC, _RDY, _CB, _SF, _SB, _SI, _SC) = range(11)


def _ft_kernel(x_ref, w_ref, o_ref):
    o_ref[...] = jnp.dot(x_ref[...], w_ref[...],
                         preferred_element_type=jnp.float32
                         ).astype(jnp.bfloat16)


def _fused_kernel(sched_ref, panel_ref, adj_ref, s1_ref, b1_ref, w2_ref,
                  b2_ref, wl1_ref, bl1_ref, wl2_ref, bl2_ref, wl3_ref,
                  bl3_ref, out_ref, buf_sc, acc_sc, t_sc, stash_sc,
                  sems, *, n, hid, bm1, bm2, cw, nc, region, stash_w, nbuf):
    t = pl.program_id(0)
    op = sched_ref[_OP, t]
    wi = sched_ref[_WI, t]
    cc = sched_ref[_CC, t]
    cb = sched_ref[_CB, t]

    @pl.when(sched_ref[_SF, t] == 1)
    def _start_next():
        src = adj_ref.at[
            pl.ds(pl.multiple_of(sched_ref[_SI, t] * bm2, bm2), bm2),
            pl.ds(pl.multiple_of(sched_ref[_SC, t] * cw, cw), cw)]
        sb = sched_ref[_SB, t]
        pltpu.make_async_copy(src, buf_sc.at[sb], sems.at[sb]).start()

    @pl.when(op == 0)
    def _phase1():
        rdy = sched_ref[_RDY, t]
        row = pl.ds(wi * bm1, bm1)
        acc_sc[row, :] = jnp.zeros((bm1, hid), jnp.float32)
        u = jnp.zeros((bm1, hid), jnp.float32)
        for c in range(nc):
            cs = c * cw
            tbc = panel_ref[:, cs:cs + cw]
            u = u + jnp.dot(tbc, s1_ref[cs:cs + cw, :],
                            preferred_element_type=jnp.float32)

            @pl.when((rdy & (1 << c)) != 0)
            def _(tbc=tbc, cs=cs):
                acc_sc[row, :] += jnp.dot(tbc, t_sc[cs:cs + cw, :],
                                          preferred_element_type=jnp.float32)
        if stash_w:
            tbt = panel_ref[:, region:n]
            u = u + jnp.dot(tbt, s1_ref[region:n, :],
                            preferred_element_type=jnp.float32)
            stash_sc[row, :] = tbt
        h1 = jnp.maximum(u + b1_ref[...], 0.0)
        t_sc[row, :] = jnp.dot(h1, w2_ref[...],
                               preferred_element_type=jnp.float32)

    @pl.when(op == 1)
    def _phase2():
        src = adj_ref.at[pl.ds(pl.multiple_of(wi * bm2, bm2), bm2),
                         pl.ds(pl.multiple_of(cc * cw, cw), cw)]
        pltpu.make_async_copy(src, buf_sc.at[cb], sems.at[cb]).wait()
        tb = buf_sc[cb]
        acc_sc[pl.ds(wi * bm2, bm2), :] += jnp.dot(
            tb, t_sc[pl.ds(pl.multiple_of(cc * cw, cw), cw), :],
            preferred_element_type=jnp.float32)

    @pl.when(op == 2)
    def _epilogue():
        row = pl.ds(wi * bm2, bm2)
        base = acc_sc[row, :]
        if stash_w:
            base = base + jnp.dot(stash_sc[row, :], t_sc[region:n, :],
                                  preferred_element_type=jnp.float32)
        h = jnp.maximum(base + b2_ref[...], 0.0)
        h = jnp.maximum(jnp.dot(h, wl1_ref[...],
                                preferred_element_type=jnp.float32)
                        + bl1_ref[...], 0.0)
        h = jnp.maximum(jnp.dot(h, wl2_ref[...],
                                preferred_element_type=jnp.float32)
                        + bl2_ref[...], 0.0)
        logits = jnp.dot(h, wl3_ref[...],
                         preferred_element_type=jnp.float32) + bl3_ref[...]
        m = jnp.max(logits, axis=-1, keepdims=True)
        lse = m + jnp.log(jnp.sum(jnp.exp(logits - m), axis=-1, keepdims=True))
        out_ref[...] = logits - lse


def _build_schedule(r1, r2, bm1, bm2, cw, nc, nbuf, depth):
    q = bm2 // bm1
    steps = []
    for i1 in range(r1):
        parent_start = bm2 * (i1 // q)
        mask = 0
        for c in range(nc):
            if (c + 1) * cw <= parent_start:
                mask |= 1 << c
        steps.append([i1, 0, 0, i1, 0, mask])
    for i2 in range(r2):
        done = bm2 * i2
        for c in range(nc):
            if (c + 1) * cw > done:
                steps.append([r1 - 1, i2, 1, i2, c, 0])
        steps.append([r1 - 1, i2, 2, i2, 0, 0])
    s = len(steps)
    sched = np.zeros((11, s), dtype=np.int32)
    for t, (pi, oi, op, wi, cc, rdy) in enumerate(steps):
        sched[_PI, t] = pi
        sched[_OI, t] = oi
        sched[_OP, t] = op
        sched[_WI, t] = wi
        sched[_CC, t] = cc
        sched[_RDY, t] = rdy
    chunk_steps = [t for t in range(s) if sched[_OP, t] == 1]
    for k, tt in enumerate(chunk_steps):
        sched[_CB, tt] = k % nbuf
        if k < depth:
            site = max(r1 - depth + k, 0)
        else:
            site = chunk_steps[k - depth]
        sched[_SF, site] = 1
        sched[_SB, site] = k % nbuf
        sched[_SI, site] = sched[_WI, tt]
        sched[_SC, site] = sched[_CC, tt]
    return jnp.asarray(sched), s


def _pick_cfg(n):
    if n % 1000 == 0 and n >= 4000:
        bm1, bm2 = 200, 1000
    else:
        bm1 = bm2 = None
        for b in (80, 40, 8):
            if n % b == 0:
                bm1 = b
                break
        bm2 = bm1
        for b in (2 * bm1, 4 * bm1):
            if n % b == 0:
                bm2 = b
    region = (n // 128) * 128
    cw = None
    for c in (768, 512, 256, 128):
        if region % c == 0 and (n < 4000) == (c <= 256):
            cw = c
            break
    if cw is None:
        cw = 128
    return bm1, bm2, cw, region


def kernel(x, adj, W_gc1, b_gc1, W_gc2, b_gc2, W_l1, b_l1, W_l2, b_l2,
           W_l3, b_l3):
    _, n, in_f = x.shape
    hid = W_gc1.shape[1]
    hid2 = W_l2.shape[1]
    classes = W_l3.shape[1]
    x2 = x.reshape(n, in_f)
    adj2 = adj.reshape(n, n)
    bm1, bm2, cw, region = _pick_cfg(n)
    r1, r2 = n // bm1, n // bm2
    nc = region // cw
    stash_w = n - region

    s1 = pl.pallas_call(
        _ft_kernel,
        grid=(r1,),
        in_specs=[
            pl.BlockSpec((bm1, in_f), lambda i: (i, 0)),
            pl.BlockSpec((in_f, hid), lambda i: (0, 0)),
        ],
        out_specs=pl.BlockSpec((bm1, hid), lambda i: (i, 0)),
        out_shape=jax.ShapeDtypeStruct((n, hid), jnp.bfloat16),
    )(x2, W_gc1)

    nbuf, depth = 6, 5
    sched, steps = _build_schedule(r1, r2, bm1, bm2, cw, nc, nbuf, depth)

    const = lambda t, s: (0, 0)
    grid_spec = pltpu.PrefetchScalarGridSpec(
        num_scalar_prefetch=1,
        grid=(steps,),
        in_specs=[
            pl.BlockSpec((bm1, n), lambda t, s: (s[_PI, t], 0)),
            pl.BlockSpec(memory_space=pl.ANY),
            pl.BlockSpec((n, hid), const),
            pl.BlockSpec((1, hid), const),
            pl.BlockSpec((hid, hid), const),
            pl.BlockSpec((1, hid), const),
            pl.BlockSpec((hid, hid), const),
            pl.BlockSpec((1, hid), const),
            pl.BlockSpec((hid, hid2), const),
            pl.BlockSpec((1, hid2), const),
            pl.BlockSpec((hid2, classes), const),
            pl.BlockSpec((1, classes), const),
        ],
        out_specs=pl.BlockSpec((bm2, classes), lambda t, s: (s[_OI, t], 0)),
        scratch_shapes=[
            pltpu.VMEM((nbuf, bm2, cw), jnp.float32),
            pltpu.VMEM((n, hid), jnp.float32),
            pltpu.VMEM((n, hid), jnp.float32),
            pltpu.VMEM((n, max(stash_w, 1)), jnp.float32),
            pltpu.SemaphoreType.DMA((nbuf,)),
        ],
    )

    out_nc = pl.pallas_call(
        functools.partial(_fused_kernel, n=n, hid=hid, bm1=bm1, bm2=bm2,
                          cw=cw, nc=nc, region=region, stash_w=stash_w,
                          nbuf=nbuf),
        grid_spec=grid_spec,
        out_shape=jax.ShapeDtypeStruct((n, classes), jnp.float32),
        compiler_params=pltpu.CompilerParams(
            dimension_semantics=("arbitrary",)),
    )(sched, adj2, adj2, s1, b_gc1.reshape(1, hid), W_gc2,
      b_gc2.reshape(1, hid), W_l1, b_l1.reshape(1, hid),
      W_l2, b_l2.reshape(1, hid2), W_l3, b_l3.reshape(1, classes))

    return jnp.transpose(out_nc)[None]


# speedup vs baseline: 1.2787x; 1.2513x over previous
import functools

import numpy as np
import jax
import jax.numpy as jnp
from jax.experimental import pallas as pl
from jax.experimental.pallas import tpu as pltpu

(_PI, _OI, _OP, _WI, _CC, _RDY, _CB, _SF, _SB, _SI, _SC) = range(11)


def _ft_kernel(x_ref, w_ref, o_ref):
    o_ref[...] = jnp.dot(x_ref[...], w_ref[...],
                         preferred_element_type=jnp.float32
                         ).astype(jnp.bfloat16)


def _fused_kernel(sched_ref, panel_ref, adj_ref, s1_ref, b1_ref, w2_ref,
                  b2_ref, wl1_ref, bl1_ref, wl2_ref, bl2_ref, wl3_ref,
                  bl3_ref, out_ref, buf_sc, acc_sc, t_sc, stash_sc,
                  sems, *, n, hid, bm1, bm2, cw, nc, region, stash_w, nbuf):
    t = pl.program_id(0)
    op = sched_ref[_OP, t]
    wi = sched_ref[_WI, t]
    cc = sched_ref[_CC, t]
    cb = sched_ref[_CB, t]

    @pl.when(sched_ref[_SF, t] == 1)
    def _start_next():
        src = adj_ref.at[
            pl.ds(pl.multiple_of(sched_ref[_SI, t] * bm2, bm2), bm2),
            pl.ds(pl.multiple_of(sched_ref[_SC, t] * cw, cw), cw)]
        sb = sched_ref[_SB, t]
        pltpu.make_async_copy(src, buf_sc.at[sb], sems.at[sb]).start()

    @pl.when(op == 0)
    def _phase1():
        rdy = sched_ref[_RDY, t]
        row = pl.ds(wi * bm1, bm1)
        acc_sc[row, :] = jnp.zeros((bm1, hid), jnp.float32)
        u = jnp.zeros((bm1, hid), jnp.float32)
        for c in range(nc):
            cs = c * cw
            tbc = panel_ref[:, cs:cs + cw]
            u = u + jnp.dot(tbc, s1_ref[cs:cs + cw, :],
                            preferred_element_type=jnp.float32)

            @pl.when((rdy & (1 << c)) != 0)
            def _(tbc=tbc, cs=cs):
                acc_sc[row, :] += jnp.dot(tbc, t_sc[cs:cs + cw, :],
                                          preferred_element_type=jnp.float32)
        if stash_w:
            tbt = panel_ref[:, region:n]
            u = u + jnp.dot(tbt, s1_ref[region:n, :],
                            preferred_element_type=jnp.float32)
            stash_sc[row, :] = tbt
        h1 = jnp.maximum(u + b1_ref[...], 0.0)
        t_sc[row, :] = jnp.dot(h1, w2_ref[...],
                               preferred_element_type=jnp.float32)

    @pl.when(op == 1)
    def _phase2():
        src = adj_ref.at[pl.ds(pl.multiple_of(wi * bm2, bm2), bm2),
                         pl.ds(pl.multiple_of(cc * cw, cw), cw)]
        pltpu.make_async_copy(src, buf_sc.at[cb], sems.at[cb]).wait()
        tb = buf_sc[cb]
        acc_sc[pl.ds(wi * bm2, bm2), :] += jnp.dot(
            tb, t_sc[pl.ds(pl.multiple_of(cc * cw, cw), cw), :],
            preferred_element_type=jnp.float32)

    @pl.when(op == 2)
    def _epilogue():
        row = pl.ds(wi * bm2, bm2)
        base = acc_sc[row, :]
        if stash_w:
            base = base + jnp.dot(stash_sc[row, :], t_sc[region:n, :],
                                  preferred_element_type=jnp.float32)
        h = jnp.maximum(base + b2_ref[...], 0.0)
        h = jnp.maximum(jnp.dot(h, wl1_ref[...],
                                preferred_element_type=jnp.float32)
                        + bl1_ref[...], 0.0)
        h = jnp.maximum(jnp.dot(h, wl2_ref[...],
                                preferred_element_type=jnp.float32)
                        + bl2_ref[...], 0.0)
        logits = jnp.dot(h, wl3_ref[...],
                         preferred_element_type=jnp.float32) + bl3_ref[...]
        m = jnp.max(logits, axis=-1, keepdims=True)
        lse = m + jnp.log(jnp.sum(jnp.exp(logits - m), axis=-1, keepdims=True))
        out_ref[...] = logits - lse


def _build_schedule(r1, r2, bm1, bm2, cw, nc, nbuf, depth):
    q = bm2 // bm1
    steps = []
    for i1 in range(r1):
        parent_start = bm2 * (i1 // q)
        mask = 0
        for c in range(nc):
            if (c + 1) * cw <= parent_start:
                mask |= 1 << c
        steps.append([i1, 0, 0, i1, 0, mask])
    for i2 in range(r2):
        done = bm2 * i2
        for c in range(nc):
            if (c + 1) * cw > done:
                steps.append([r1 - 1, i2, 1, i2, c, 0])
        steps.append([r1 - 1, i2, 2, i2, 0, 0])
    s = len(steps)
    sched = np.zeros((11, s), dtype=np.int32)
    for t, (pi, oi, op, wi, cc, rdy) in enumerate(steps):
        sched[_PI, t] = pi
        sched[_OI, t] = oi
        sched[_OP, t] = op
        sched[_WI, t] = wi
        sched[_CC, t] = cc
        sched[_RDY, t] = rdy
    chunk_steps = [t for t in range(s) if sched[_OP, t] == 1]
    for k, tt in enumerate(chunk_steps):
        sched[_CB, tt] = k % nbuf
        if k < depth:
            site = max(r1 - depth + k, 0)
        else:
            site = chunk_steps[k - depth]
        sched[_SF, site] = 1
        sched[_SB, site] = k % nbuf
        sched[_SI, site] = sched[_WI, tt]
        sched[_SC, site] = sched[_CC, tt]
    return jnp.asarray(sched), s


def _pick_cfg(n):
    if n % 200 == 0:
        bm1 = bm2 = 200
    else:
        bm1 = bm2 = None
        for b in (80, 40, 8):
            if n % b == 0:
                bm1 = bm2 = b
                break
    region = (n // 128) * 128
    cw = 128
    for c in (3328, 1664, 768, 512, 384, 256):
        if region % c == 0:
            cw = c
            break
    return bm1, bm2, cw, region


def kernel(x, adj, W_gc1, b_gc1, W_gc2, b_gc2, W_l1, b_l1, W_l2, b_l2,
           W_l3, b_l3):
    _, n, in_f = x.shape
    hid = W_gc1.shape[1]
    hid2 = W_l2.shape[1]
    classes = W_l3.shape[1]
    x2 = x.reshape(n, in_f)
    adj2 = adj.reshape(n, n)
    bm1, bm2, cw, region = _pick_cfg(n)
    r1, r2 = n // bm1, n // bm2
    nc = region // cw
    stash_w = n - region

    s1 = pl.pallas_call(
        _ft_kernel,
        grid=(r1,),
        in_specs=[
            pl.BlockSpec((bm1, in_f), lambda i: (i, 0)),
            pl.BlockSpec((in_f, hid), lambda i: (0, 0)),
        ],
        out_specs=pl.BlockSpec((bm1, hid), lambda i: (i, 0)),
        out_shape=jax.ShapeDtypeStruct((n, hid), jnp.bfloat16),
    )(x2, W_gc1)

    nbuf, depth = 5, 4
    sched, steps = _build_schedule(r1, r2, bm1, bm2, cw, nc, nbuf, depth)

    const = lambda t, s: (0, 0)
    grid_spec = pltpu.PrefetchScalarGridSpec(
        num_scalar_prefetch=1,
        grid=(steps,),
        in_specs=[
            pl.BlockSpec((bm1, n), lambda t, s: (s[_PI, t], 0)),
            pl.BlockSpec(memory_space=pl.ANY),
            pl.BlockSpec((n, hid), const),
            pl.BlockSpec((1, hid), const),
            pl.BlockSpec((hid, hid), const),
            pl.BlockSpec((1, hid), const),
            pl.BlockSpec((hid, hid), const),
            pl.BlockSpec((1, hid), const),
            pl.BlockSpec((hid, hid2), const),
            pl.BlockSpec((1, hid2), const),
            pl.BlockSpec((hid2, classes), const),
            pl.BlockSpec((1, classes), const),
        ],
        out_specs=pl.BlockSpec((bm2, classes), lambda t, s: (s[_OI, t], 0)),
        scratch_shapes=[
            pltpu.VMEM((nbuf, bm2, cw), jnp.float32),
            pltpu.VMEM((n, hid), jnp.float32),
            pltpu.VMEM((n, hid), jnp.float32),
            pltpu.VMEM((n, max(stash_w, 1)), jnp.float32),
            pltpu.SemaphoreType.DMA((nbuf,)),
        ],
    )

    out_nc = pl.pallas_call(
        functools.partial(_fused_kernel, n=n, hid=hid, bm1=bm1, bm2=bm2,
                          cw=cw, nc=nc, region=region, stash_w=stash_w,
                          nbuf=nbuf),
        grid_spec=grid_spec,
        out_shape=jax.ShapeDtypeStruct((n, classes), jnp.float32),
        compiler_params=pltpu.CompilerParams(
            dimension_semantics=("arbitrary",)),
    )(sched, adj2, adj2, s1, b_gc1.reshape(1, hid), W_gc2,
      b_gc2.reshape(1, hid), W_l1, b_l1.reshape(1, hid),
      W_l2, b_l2.reshape(1, hid2), W_l3, b_l3.reshape(1, classes))

    return jnp.transpose(out_nc)[None]
